# initial kernel scaffold (unmeasured)
import jax
import jax.numpy as jnp
from jax import lax
from jax.experimental import pallas as pl
from jax.experimental.pallas import tpu as pltpu

N_DEV = 16
M = 4096
N = 8192
CW = N // N_DEV


def _ar_body(part_ref, scale_ref, out_ref,
             send_buf, recv_buf, ld_buf, acc_buf,
             send_sems, recv_sems, ld_sem, st_sems,
             ag_send_sems, ag_recv_sems,
             rs_credit, ag_credit):
    me = lax.axis_index("i")
    right = lax.rem(me + 1, N_DEV)
    left = lax.rem(me + N_DEV - 1, N_DEV)

    barrier = pltpu.get_barrier_semaphore()
    for nbr in (left, right):
        pl.semaphore_signal(barrier, inc=1, device_id=(nbr,),
                            device_id_type=pl.DeviceIdType.MESH)
    pl.semaphore_wait(barrier, 2)

    def cs(c):
        return pl.ds(c * CW, CW)

    cp = pltpu.make_async_copy(part_ref.at[:, cs(me)], send_buf.at[0], ld_sem)
    cp.start()
    cp.wait()

    for s in range(N_DEV - 1):
        slot = s % 2
        if s >= 2:
            pl.semaphore_wait(rs_credit, 1)
        rdma = pltpu.make_async_remote_copy(
            src_ref=send_buf.at[slot],
            dst_ref=recv_buf.at[slot],
            send_sem=send_sems.at[slot],
            recv_sem=recv_sems.at[slot],
            device_id=(right,),
            device_id_type=pl.DeviceIdType.MESH,
        )
        rdma.start()
        c = lax.rem(me - (s + 1) + N_DEV, N_DEV)
        ld = pltpu.make_async_copy(part_ref.at[:, cs(c)], ld_buf, ld_sem)
        ld.start()
        ld.wait()
        rdma.wait()
        if s < N_DEV - 2:
            send_buf[(s + 1) % 2] = recv_buf[slot] + ld_buf[...]
        else:
            acc_buf[...] = recv_buf[slot] + ld_buf[...]
        if s <= N_DEV - 4:
            pl.semaphore_signal(rs_credit, inc=1, device_id=(left,),
                                device_id_type=pl.DeviceIdType.MESH)

    cmine = lax.rem(me + 1, N_DEV)
    y = acc_buf[...] * scale_ref[0, 0]
    sg = 1.0 / (1.0 + jnp.exp(-jnp.clip(y, -60.0, 60.0)))
    send_buf[0] = y * sg

    st = pltpu.make_async_copy(send_buf.at[0], out_ref.at[:, cs(cmine)],
                               st_sems.at[1])
    st.start()

    for t in range(N_DEV - 1):
        if t >= 1:
            pl.semaphore_wait(ag_credit, 1)
        rdma = pltpu.make_async_remote_copy(
            src_ref=send_buf.at[t % 2],
            dst_ref=send_buf.at[(t + 1) % 2],
            send_sem=ag_send_sems.at[t % 2],
            recv_sem=ag_recv_sems.at[(t + 1) % 2],
            device_id=(right,),
            device_id_type=pl.DeviceIdType.MESH,
        )
        rdma.start()
        rdma.wait()
        r = lax.rem(me - t + N_DEV, N_DEV)
        st = pltpu.make_async_copy(send_buf.at[(t + 1) % 2],
                                   out_ref.at[:, cs(r)], st_sems.at[t % 2])
        st.start()
        pltpu.make_async_copy(send_buf.at[t % 2],
                              out_ref.at[:, cs(r)],
                              st_sems.at[(t + 1) % 2]).wait()
        if t <= N_DEV - 3:
            pl.semaphore_signal(ag_credit, inc=1, device_id=(left,),
                                device_id_type=pl.DeviceIdType.MESH)
    pltpu.make_async_copy(send_buf.at[0], out_ref.at[:, cs(me)],
                          st_sems.at[0]).wait()


def kernel(x, w_mat, scale_x, scale_w):
    partial = jnp.dot(x.astype(jnp.bfloat16), w_mat.astype(jnp.bfloat16),
                      preferred_element_type=jnp.float32)
    scale = jnp.reshape(scale_x.astype(jnp.float32)
                        * scale_w.astype(jnp.float32), (1, 1))

    return pl.pallas_call(
        _ar_body,
        out_shape=jax.ShapeDtypeStruct((M, N), jnp.float32),
        in_specs=[
            pl.BlockSpec(memory_space=pltpu.ANY),
            pl.BlockSpec(memory_space=pltpu.SMEM),
        ],
        out_specs=pl.BlockSpec(memory_space=pltpu.ANY),
        scratch_shapes=[
            pltpu.VMEM((2, M, CW), jnp.float32),
            pltpu.VMEM((2, M, CW), jnp.float32),
            pltpu.VMEM((M, CW), jnp.float32),
            pltpu.VMEM((M, CW), jnp.float32),
            pltpu.SemaphoreType.DMA((2,)),
            pltpu.SemaphoreType.DMA((2,)),
            pltpu.SemaphoreType.DMA,
            pltpu.SemaphoreType.DMA((2,)),
            pltpu.SemaphoreType.DMA((2,)),
            pltpu.SemaphoreType.DMA((2,)),
            pltpu.SemaphoreType.REGULAR,
            pltpu.SemaphoreType.REGULAR,
        ],
        compiler_params=pltpu.CompilerParams(collective_id=0),
    )(partial, scale)


# baseline (device time: 3061330 ns/iter reference)
import jax
import jax.numpy as jnp
from jax import lax
from jax.experimental import pallas as pl
from jax.experimental.pallas import tpu as pltpu

N_DEV = 16
M = 4096
N = 8192
CW = N // N_DEV
HROW = M // 2


def _ar_body(part_ref, scale_ref, out_ref,
             send_buf, recv_buf, ld_buf, acc_buf,
             send_sems, recv_sems, ld_sem, st_sems,
             ag_send_sems, ag_recv_sems,
             rs_credit, ag_credit):
    me = lax.axis_index("i")
    right = lax.rem(me + 1, N_DEV)
    left = lax.rem(me + N_DEV - 1, N_DEV)

    barrier = pltpu.get_barrier_semaphore()
    for nbr in (left, right):
        pl.semaphore_signal(barrier, inc=1, device_id=(nbr,),
                            device_id_type=pl.DeviceIdType.MESH)
    pl.semaphore_wait(barrier, 2)

    for h in range(2):
        ro = h * HROW

        def blk(ref, c):
            return ref.at[pl.ds(ro, HROW), pl.ds(c * CW, CW)]

        cp = pltpu.make_async_copy(blk(part_ref, me), send_buf.at[0], ld_sem)
        cp.start()
        cp.wait()

        for s in range(N_DEV - 1):
            slot = s % 2
            if s >= 2:
                pl.semaphore_wait(rs_credit, 1)
            rdma = pltpu.make_async_remote_copy(
                src_ref=send_buf.at[slot],
                dst_ref=recv_buf.at[slot],
                send_sem=send_sems.at[slot],
                recv_sem=recv_sems.at[slot],
                device_id=(right,),
                device_id_type=pl.DeviceIdType.MESH,
            )
            rdma.start()
            c = lax.rem(me - (s + 1) + N_DEV, N_DEV)
            ld = pltpu.make_async_copy(blk(part_ref, c), ld_buf, ld_sem)
            ld.start()
            ld.wait()
            rdma.wait()
            if s < N_DEV - 2:
                send_buf[(s + 1) % 2] = recv_buf[slot] + ld_buf[...]
            else:
                acc_buf[...] = recv_buf[slot] + ld_buf[...]
            if s <= N_DEV - 4:
                pl.semaphore_signal(rs_credit, inc=1, device_id=(left,),
                                    device_id_type=pl.DeviceIdType.MESH)

        cmine = lax.rem(me + 1, N_DEV)
        y = acc_buf[...] * scale_ref[0, 0]
        sg = 1.0 / (1.0 + jnp.exp(-jnp.clip(y, -60.0, 60.0)))
        send_buf[0] = y * sg

        st = pltpu.make_async_copy(send_buf.at[0], blk(out_ref, cmine),
                                   st_sems.at[1])
        st.start()

        for t in range(N_DEV - 1):
            if t >= 1:
                pl.semaphore_wait(ag_credit, 1)
            rdma = pltpu.make_async_remote_copy(
                src_ref=send_buf.at[t % 2],
                dst_ref=send_buf.at[(t + 1) % 2],
                send_sem=ag_send_sems.at[t % 2],
                recv_sem=ag_recv_sems.at[(t + 1) % 2],
                device_id=(right,),
                device_id_type=pl.DeviceIdType.MESH,
            )
            rdma.start()
            rdma.wait()
            r = lax.rem(me - t + N_DEV, N_DEV)
            st = pltpu.make_async_copy(send_buf.at[(t + 1) % 2],
                                       blk(out_ref, r), st_sems.at[t % 2])
            st.start()
            pltpu.make_async_copy(send_buf.at[t % 2], blk(out_ref, r),
                                  st_sems.at[(t + 1) % 2]).wait()
            if t <= N_DEV - 3:
                pl.semaphore_signal(ag_credit, inc=1, device_id=(left,),
                                    device_id_type=pl.DeviceIdType.MESH)
        pltpu.make_async_copy(send_buf.at[0], blk(out_ref, me),
                              st_sems.at[0]).wait()


def kernel(x, w_mat, scale_x, scale_w):
    partial = jnp.dot(x.astype(jnp.bfloat16), w_mat.astype(jnp.bfloat16),
                      preferred_element_type=jnp.float32)
    scale = jnp.reshape(scale_x.astype(jnp.float32)
                        * scale_w.astype(jnp.float32), (1, 1))

    return pl.pallas_call(
        _ar_body,
        out_shape=jax.ShapeDtypeStruct((M, N), jnp.float32),
        in_specs=[
            pl.BlockSpec(memory_space=pl.ANY),
            pl.BlockSpec(memory_space=pltpu.MemorySpace.SMEM),
        ],
        out_specs=pl.BlockSpec(memory_space=pl.ANY),
        scratch_shapes=[
            pltpu.VMEM((2, HROW, CW), jnp.float32),
            pltpu.VMEM((2, HROW, CW), jnp.float32),
            pltpu.VMEM((HROW, CW), jnp.float32),
            pltpu.VMEM((HROW, CW), jnp.float32),
            pltpu.SemaphoreType.DMA((2,)),
            pltpu.SemaphoreType.DMA((2,)),
            pltpu.SemaphoreType.DMA,
            pltpu.SemaphoreType.DMA((2,)),
            pltpu.SemaphoreType.DMA((2,)),
            pltpu.SemaphoreType.DMA((2,)),
            pltpu.SemaphoreType.REGULAR,
            pltpu.SemaphoreType.REGULAR,
        ],
        compiler_params=pltpu.CompilerParams(collective_id=0),
    )(partial, scale)


# device time: 1713601 ns/iter; 1.7865x vs baseline; 1.7865x over previous
import jax
import jax.numpy as jnp
from jax import lax
from jax.experimental import pallas as pl
from jax.experimental.pallas import tpu as pltpu

N_DEV = 16
M = 4096
N = 8192
CW = N // N_DEV
HROW = M // 2


def _ar_body(part_ref, scale_ref, out_ref,
             send_buf, recv_buf, ld_buf, acc_buf, stg_buf,
             send_sems, recv_sems, ld_sem, st_sems,
             ag_send_sems, ag_recv_sems,
             rs_credit, ag_credit):
    me = lax.axis_index("i")
    right = lax.rem(me + 1, N_DEV)
    left = lax.rem(me + N_DEV - 1, N_DEV)

    barrier = pltpu.get_barrier_semaphore()
    for nbr in (left, right):
        pl.semaphore_signal(barrier, inc=1, device_id=(nbr,),
                            device_id_type=pl.DeviceIdType.MESH)
    pl.semaphore_wait(barrier, 2)

    for h in range(2):
        ro = h * HROW

        def blk(ref, c):
            return ref.at[pl.ds(ro, HROW), pl.ds(c * CW, CW)]

        cp = pltpu.make_async_copy(blk(part_ref, me), ld_buf, ld_sem)
        cp.start()
        cp.wait()
        send_buf[0] = ld_buf[...].astype(jnp.bfloat16)

        for s in range(N_DEV - 1):
            slot = s % 2
            if s >= 2:
                pl.semaphore_wait(rs_credit, 1)
            rdma = pltpu.make_async_remote_copy(
                src_ref=send_buf.at[slot],
                dst_ref=recv_buf.at[slot],
                send_sem=send_sems.at[slot],
                recv_sem=recv_sems.at[slot],
                device_id=(right,),
                device_id_type=pl.DeviceIdType.MESH,
            )
            rdma.start()
            c = lax.rem(me - (s + 1) + N_DEV, N_DEV)
            ld = pltpu.make_async_copy(blk(part_ref, c), ld_buf, ld_sem)
            ld.start()
            ld.wait()
            rdma.wait()
            if s < N_DEV - 2:
                send_buf[(s + 1) % 2] = (
                    recv_buf[slot].astype(jnp.float32) + ld_buf[...]
                ).astype(jnp.bfloat16)
            else:
                acc_buf[...] = recv_buf[slot].astype(jnp.float32) + ld_buf[...]
            if s <= N_DEV - 4:
                pl.semaphore_signal(rs_credit, inc=1, device_id=(left,),
                                    device_id_type=pl.DeviceIdType.MESH)

        cmine = lax.rem(me + 1, N_DEV)
        y = acc_buf[...] * scale_ref[0, 0]
        sg = 1.0 / (1.0 + jnp.exp(-jnp.clip(y, -60.0, 60.0)))
        stg_buf[1] = y * sg
        send_buf[0] = stg_buf[1].astype(jnp.bfloat16)

        st = pltpu.make_async_copy(stg_buf.at[1], blk(out_ref, cmine),
                                   st_sems.at[1])
        st.start()

        for t in range(N_DEV - 1):
            if t >= 1:
                pl.semaphore_wait(ag_credit, 1)
            rdma = pltpu.make_async_remote_copy(
                src_ref=send_buf.at[t % 2],
                dst_ref=send_buf.at[(t + 1) % 2],
                send_sem=ag_send_sems.at[t % 2],
                recv_sem=ag_recv_sems.at[(t + 1) % 2],
                device_id=(right,),
                device_id_type=pl.DeviceIdType.MESH,
            )
            rdma.start()
            rdma.wait()
            if t <= N_DEV - 3:
                pl.semaphore_signal(ag_credit, inc=1, device_id=(left,),
                                    device_id_type=pl.DeviceIdType.MESH)
            r = lax.rem(me - t + N_DEV, N_DEV)
            if t >= 1:
                pltpu.make_async_copy(stg_buf.at[t % 2], blk(out_ref, r),
                                      st_sems.at[t % 2]).wait()
            stg_buf[t % 2] = send_buf[(t + 1) % 2].astype(jnp.float32)
            st = pltpu.make_async_copy(stg_buf.at[t % 2], blk(out_ref, r),
                                       st_sems.at[t % 2])
            st.start()
        pltpu.make_async_copy(stg_buf.at[1], blk(out_ref, me),
                              st_sems.at[1]).wait()
        pltpu.make_async_copy(stg_buf.at[0], blk(out_ref, me),
                              st_sems.at[0]).wait()


def kernel(x, w_mat, scale_x, scale_w):
    partial = jnp.dot(x.astype(jnp.bfloat16), w_mat.astype(jnp.bfloat16),
                      preferred_element_type=jnp.float32)
    scale = jnp.reshape(scale_x.astype(jnp.float32)
                        * scale_w.astype(jnp.float32), (1, 1))

    return pl.pallas_call(
        _ar_body,
        out_shape=jax.ShapeDtypeStruct((M, N), jnp.float32),
        in_specs=[
            pl.BlockSpec(memory_space=pl.ANY),
            pl.BlockSpec(memory_space=pltpu.MemorySpace.SMEM),
        ],
        out_specs=pl.BlockSpec(memory_space=pl.ANY),
        scratch_shapes=[
            pltpu.VMEM((2, HROW, CW), jnp.bfloat16),
            pltpu.VMEM((2, HROW, CW), jnp.bfloat16),
            pltpu.VMEM((HROW, CW), jnp.float32),
            pltpu.VMEM((HROW, CW), jnp.float32),
            pltpu.VMEM((2, HROW, CW), jnp.float32),
            pltpu.SemaphoreType.DMA((2,)),
            pltpu.SemaphoreType.DMA((2,)),
            pltpu.SemaphoreType.DMA,
            pltpu.SemaphoreType.DMA((2,)),
            pltpu.SemaphoreType.DMA((2,)),
            pltpu.SemaphoreType.DMA((2,)),
            pltpu.SemaphoreType.REGULAR,
            pltpu.SemaphoreType.REGULAR,
        ],
        compiler_params=pltpu.CompilerParams(collective_id=0),
    )(partial, scale)


# device time: 1076450 ns/iter; 2.8439x vs baseline; 1.5919x over previous
import jax
import jax.numpy as jnp
from jax import lax
from jax.experimental import pallas as pl
from jax.experimental.pallas import tpu as pltpu

N_DEV = 16
M = 4096
N = 8192
CW = N // N_DEV
CW2 = CW // 2
HROW = M // 2


class _Lane:
    def __init__(self, me, sgn, half, to, frm, send, recv, ld, acc, stg,
                 ssem, rsem, ldsem, stsem, agss, agrs, rscred, agcred):
        self.me, self.sgn, self.half = me, sgn, half
        self.to, self.frm = to, frm
        self.send, self.recv, self.ld, self.acc, self.stg = (
            send, recv, ld, acc, stg)
        self.ssem, self.rsem, self.ldsem, self.stsem = ssem, rsem, ldsem, stsem
        self.agss, self.agrs, self.rscred, self.agcred = (
            agss, agrs, rscred, agcred)

    def chunk(self, k):
        return lax.rem(self.me + self.sgn * k + 2 * N_DEV, N_DEV)


def _ar_body(part_ref, scale_ref, out_ref,
             send_r, recv_r, ld_r, acc_r, stg_r,
             send_l, recv_l, ld_l, acc_l, stg_l,
             ssem_r, rsem_r, ldsem_r, stsem_r, agss_r, agrs_r,
             ssem_l, rsem_l, ldsem_l, stsem_l, agss_l, agrs_l,
             rscred_r, agcred_r, rscred_l, agcred_l):
    me = lax.axis_index("i")
    right = lax.rem(me + 1, N_DEV)
    left = lax.rem(me + N_DEV - 1, N_DEV)

    barrier = pltpu.get_barrier_semaphore()
    for nbr in (left, right):
        pl.semaphore_signal(barrier, inc=1, device_id=(nbr,),
                            device_id_type=pl.DeviceIdType.MESH)
    pl.semaphore_wait(barrier, 2)

    lanes = (
        _Lane(me, -1, 0, right, left, send_r, recv_r, ld_r, acc_r, stg_r,
              ssem_r, rsem_r, ldsem_r, stsem_r, agss_r, agrs_r,
              rscred_r, agcred_r),
        _Lane(me, +1, 1, left, right, send_l, recv_l, ld_l, acc_l, stg_l,
              ssem_l, rsem_l, ldsem_l, stsem_l, agss_l, agrs_l,
              rscred_l, agcred_l),
    )

    for h in range(2):
        ro = h * HROW

        def blk(c, half):
            return part_ref.at[pl.ds(ro, HROW),
                               pl.ds(c * CW + half * CW2, CW2)]

        def oblk(c, half):
            return out_ref.at[pl.ds(ro, HROW),
                              pl.ds(c * CW + half * CW2, CW2)]

        for ln in lanes:
            pltpu.make_async_copy(blk(me, ln.half), ln.ld, ln.ldsem).start()
        for ln in lanes:
            pltpu.make_async_copy(blk(me, ln.half), ln.ld, ln.ldsem).wait()
            ln.send[0] = ln.ld[...].astype(jnp.bfloat16)

        for s in range(N_DEV - 1):
            slot = s % 2
            rdmas = []
            for ln in lanes:
                if s >= 2:
                    pl.semaphore_wait(ln.rscred, 1)
                rdma = pltpu.make_async_remote_copy(
                    src_ref=ln.send.at[slot],
                    dst_ref=ln.recv.at[slot],
                    send_sem=ln.ssem.at[slot],
                    recv_sem=ln.rsem.at[slot],
                    device_id=(ln.to,),
                    device_id_type=pl.DeviceIdType.MESH,
                )
                rdma.start()
                rdmas.append(rdma)
            for ln in lanes:
                pltpu.make_async_copy(
                    blk(ln.chunk(s + 1), ln.half), ln.ld, ln.ldsem).start()
            for ln, rdma in zip(lanes, rdmas):
                pltpu.make_async_copy(
                    blk(ln.chunk(s + 1), ln.half), ln.ld, ln.ldsem).wait()
                rdma.wait()
                if s < N_DEV - 2:
                    ln.send[(s + 1) % 2] = (
                        ln.recv[slot].astype(jnp.float32) + ln.ld[...]
                    ).astype(jnp.bfloat16)
                else:
                    ln.acc[...] = ln.recv[slot].astype(jnp.float32) + ln.ld[...]
                if s <= N_DEV - 4:
                    pl.semaphore_signal(ln.rscred, inc=1, device_id=(ln.frm,),
                                        device_id_type=pl.DeviceIdType.MESH)

        for ln in lanes:
            own = ln.chunk(N_DEV - 1)
            y = ln.acc[...] * scale_ref[0, 0]
            sg = 1.0 / (1.0 + jnp.exp(-jnp.clip(y, -60.0, 60.0)))
            ln.stg[1] = y * sg
            ln.send[0] = ln.stg[1].astype(jnp.bfloat16)
            pltpu.make_async_copy(ln.stg.at[1], oblk(own, ln.half),
                                  ln.stsem.at[1]).start()

        for t in range(N_DEV - 1):
            rdmas = []
            for ln in lanes:
                if t >= 1:
                    pl.semaphore_wait(ln.agcred, 1)
                rdma = pltpu.make_async_remote_copy(
                    src_ref=ln.send.at[t % 2],
                    dst_ref=ln.send.at[(t + 1) % 2],
                    send_sem=ln.agss.at[t % 2],
                    recv_sem=ln.agrs.at[(t + 1) % 2],
                    device_id=(ln.to,),
                    device_id_type=pl.DeviceIdType.MESH,
                )
                rdma.start()
                rdmas.append(rdma)
            for ln, rdma in zip(lanes, rdmas):
                rdma.wait()
                if t <= N_DEV - 3:
                    pl.semaphore_signal(ln.agcred, inc=1, device_id=(ln.frm,),
                                        device_id_type=pl.DeviceIdType.MESH)
                r = ln.chunk(t)
                if t >= 1:
                    pltpu.make_async_copy(ln.stg.at[t % 2], oblk(r, ln.half),
                                          ln.stsem.at[t % 2]).wait()
                ln.stg[t % 2] = ln.send[(t + 1) % 2].astype(jnp.float32)
                pltpu.make_async_copy(ln.stg.at[t % 2], oblk(r, ln.half),
                                      ln.stsem.at[t % 2]).start()
        for ln in lanes:
            for sl in (1, 0):
                pltpu.make_async_copy(ln.stg.at[sl], oblk(me, ln.half),
                                      ln.stsem.at[sl]).wait()


def kernel(x, w_mat, scale_x, scale_w):
    partial = jnp.dot(x.astype(jnp.bfloat16), w_mat.astype(jnp.bfloat16),
                      preferred_element_type=jnp.float32)
    scale = jnp.reshape(scale_x.astype(jnp.float32)
                        * scale_w.astype(jnp.float32), (1, 1))

    lane_bufs = [
        pltpu.VMEM((2, HROW, CW2), jnp.bfloat16),
        pltpu.VMEM((2, HROW, CW2), jnp.bfloat16),
        pltpu.VMEM((HROW, CW2), jnp.float32),
        pltpu.VMEM((HROW, CW2), jnp.float32),
        pltpu.VMEM((2, HROW, CW2), jnp.float32),
    ]
    lane_sems = [
        pltpu.SemaphoreType.DMA((2,)),
        pltpu.SemaphoreType.DMA((2,)),
        pltpu.SemaphoreType.DMA,
        pltpu.SemaphoreType.DMA((2,)),
        pltpu.SemaphoreType.DMA((2,)),
        pltpu.SemaphoreType.DMA((2,)),
    ]
    credits = [pltpu.SemaphoreType.REGULAR] * 4

    return pl.pallas_call(
        _ar_body,
        out_shape=jax.ShapeDtypeStruct((M, N), jnp.float32),
        in_specs=[
            pl.BlockSpec(memory_space=pl.ANY),
            pl.BlockSpec(memory_space=pltpu.MemorySpace.SMEM),
        ],
        out_specs=pl.BlockSpec(memory_space=pl.ANY),
        scratch_shapes=lane_bufs + lane_bufs + lane_sems + lane_sems + credits,
        compiler_params=pltpu.CompilerParams(collective_id=0),
    )(partial, scale)


# device time: 1018521 ns/iter; 3.0057x vs baseline; 1.0569x over previous
import jax
import jax.numpy as jnp
from jax import lax
from jax.experimental import pallas as pl
from jax.experimental.pallas import tpu as pltpu

N_DEV = 16
M = 4096
N = 8192
CW = N // N_DEV
CW2 = CW // 2
HROW = M // 2
SROW = HROW // 2
N_LANE = 4


class _Lane:
    def __init__(self, me, sgn, half, rsub, to, frm, bufs, sems, creds):
        self.me, self.sgn, self.half, self.rsub = me, sgn, half, rsub
        self.to, self.frm = to, frm
        self.send, self.recv, self.ld, self.acc, self.stg = bufs
        (self.ssem, self.rsem, self.ldsem, self.stsem,
         self.agss, self.agrs) = sems
        self.rscred, self.agcred = creds

    def chunk(self, k):
        return lax.rem(self.me + self.sgn * k + 2 * N_DEV, N_DEV)


def _ar_body(part_ref, scale_ref, out_ref, *scr):
    me = lax.axis_index("i")
    right = lax.rem(me + 1, N_DEV)
    left = lax.rem(me + N_DEV - 1, N_DEV)
    ro = pl.program_id(0) * HROW

    barrier = pltpu.get_barrier_semaphore()
    for nbr in (left, right):
        pl.semaphore_signal(barrier, inc=1, device_id=(nbr,),
                            device_id_type=pl.DeviceIdType.MESH)
    pl.semaphore_wait(barrier, 2)

    cfg = ((-1, 0, 0), (+1, 1, 0), (-1, 0, 1), (+1, 1, 1))
    lanes = tuple(
        _Lane(me, sgn, half, rsub,
              right if sgn < 0 else left,
              left if sgn < 0 else right,
              scr[5 * i: 5 * i + 5],
              scr[20 + 6 * i: 20 + 6 * i + 6],
              scr[44 + 2 * i: 44 + 2 * i + 2])
        for i, (sgn, half, rsub) in enumerate(cfg)
    )

    def blk(c, ln):
        return part_ref.at[pl.ds(ro + ln.rsub * SROW, SROW),
                           pl.ds(c * CW + ln.half * CW2, CW2)]

    def oblk(c, ln):
        return out_ref.at[pl.ds(ro + ln.rsub * SROW, SROW),
                          pl.ds(c * CW + ln.half * CW2, CW2)]

    for ln in lanes:
        pltpu.make_async_copy(blk(me, ln), ln.ld, ln.ldsem).start()
    for ln in lanes:
        pltpu.make_async_copy(blk(me, ln), ln.ld, ln.ldsem).wait()
        ln.send[0] = ln.ld[...].astype(jnp.bfloat16)

    for s in range(N_DEV - 1):
        slot = s % 2
        rdmas = []
        for ln in lanes:
            if s >= 2:
                pl.semaphore_wait(ln.rscred, 1)
            rdma = pltpu.make_async_remote_copy(
                src_ref=ln.send.at[slot],
                dst_ref=ln.recv.at[slot],
                send_sem=ln.ssem.at[slot],
                recv_sem=ln.rsem.at[slot],
                device_id=(ln.to,),
                device_id_type=pl.DeviceIdType.MESH,
            )
            rdma.start()
            rdmas.append(rdma)
        for ln in lanes:
            pltpu.make_async_copy(blk(ln.chunk(s + 1), ln),
                                  ln.ld, ln.ldsem).start()
        for ln, rdma in zip(lanes, rdmas):
            pltpu.make_async_copy(blk(ln.chunk(s + 1), ln),
                                  ln.ld, ln.ldsem).wait()
            rdma.wait()
            if s < N_DEV - 2:
                ln.send[(s + 1) % 2] = (
                    ln.recv[slot].astype(jnp.float32) + ln.ld[...]
                ).astype(jnp.bfloat16)
            else:
                ln.acc[...] = ln.recv[slot].astype(jnp.float32) + ln.ld[...]
            if s <= N_DEV - 4:
                pl.semaphore_signal(ln.rscred, inc=1, device_id=(ln.frm,),
                                    device_id_type=pl.DeviceIdType.MESH)

    for ln in lanes:
        own = ln.chunk(N_DEV - 1)
        y = ln.acc[...] * scale_ref[0, 0]
        sg = 1.0 / (1.0 + jnp.exp(-jnp.clip(y, -60.0, 60.0)))
        ln.stg[1] = y * sg
        ln.send[0] = ln.stg[1].astype(jnp.bfloat16)
        pltpu.make_async_copy(ln.stg.at[1], oblk(own, ln),
                              ln.stsem.at[1]).start()

    for t in range(N_DEV - 1):
        rdmas = []
        for ln in lanes:
            if t >= 1:
                pl.semaphore_wait(ln.agcred, 1)
            rdma = pltpu.make_async_remote_copy(
                src_ref=ln.send.at[t % 2],
                dst_ref=ln.send.at[(t + 1) % 2],
                send_sem=ln.agss.at[t % 2],
                recv_sem=ln.agrs.at[(t + 1) % 2],
                device_id=(ln.to,),
                device_id_type=pl.DeviceIdType.MESH,
            )
            rdma.start()
            rdmas.append(rdma)
        for ln, rdma in zip(lanes, rdmas):
            rdma.wait()
            if t <= N_DEV - 3:
                pl.semaphore_signal(ln.agcred, inc=1, device_id=(ln.frm,),
                                    device_id_type=pl.DeviceIdType.MESH)
            r = ln.chunk(t)
            if t >= 1:
                pltpu.make_async_copy(ln.stg.at[t % 2], oblk(r, ln),
                                      ln.stsem.at[t % 2]).wait()
            ln.stg[t % 2] = ln.send[(t + 1) % 2].astype(jnp.float32)
            pltpu.make_async_copy(ln.stg.at[t % 2], oblk(r, ln),
                                  ln.stsem.at[t % 2]).start()
    for ln in lanes:
        for sl in (1, 0):
            pltpu.make_async_copy(ln.stg.at[sl], oblk(me, ln),
                                  ln.stsem.at[sl]).wait()


def kernel(x, w_mat, scale_x, scale_w):
    partial = jnp.dot(x.astype(jnp.bfloat16), w_mat.astype(jnp.bfloat16),
                      preferred_element_type=jnp.float32)
    scale = jnp.reshape(scale_x.astype(jnp.float32)
                        * scale_w.astype(jnp.float32), (1, 1))

    lane_bufs = [
        pltpu.VMEM((2, SROW, CW2), jnp.bfloat16),
        pltpu.VMEM((2, SROW, CW2), jnp.bfloat16),
        pltpu.VMEM((SROW, CW2), jnp.float32),
        pltpu.VMEM((SROW, CW2), jnp.float32),
        pltpu.VMEM((2, SROW, CW2), jnp.float32),
    ]
    lane_sems = [
        pltpu.SemaphoreType.DMA((2,)),
        pltpu.SemaphoreType.DMA((2,)),
        pltpu.SemaphoreType.DMA,
        pltpu.SemaphoreType.DMA((2,)),
        pltpu.SemaphoreType.DMA((2,)),
        pltpu.SemaphoreType.DMA((2,)),
    ]
    scratch = (lane_bufs * N_LANE
               + lane_sems * N_LANE
               + [pltpu.SemaphoreType.REGULAR] * (2 * N_LANE))

    return pl.pallas_call(
        _ar_body,
        grid=(2,),
        out_shape=jax.ShapeDtypeStruct((M, N), jnp.float32),
        in_specs=[
            pl.BlockSpec(memory_space=pl.ANY),
            pl.BlockSpec(memory_space=pltpu.MemorySpace.SMEM),
        ],
        out_specs=pl.BlockSpec(memory_space=pl.ANY),
        scratch_shapes=scratch,
        compiler_params=pltpu.CompilerParams(collective_id=0),
    )(partial, scale)


# device time: 839345 ns/iter; 3.6473x vs baseline; 1.2135x over previous
import jax
import jax.numpy as jnp
from jax import lax
from jax.experimental import pallas as pl
from jax.experimental.pallas import tpu as pltpu

N_DEV = 16
M = 4096
N = 8192
CW = N // N_DEV
CW2 = CW // 2
HROW = M // 2
SROW = HROW // 2
N_LANE = 4


class _Lane:
    def __init__(self, me, sgn, half, rsub, to, frm, bufs, sems, creds):
        self.me, self.sgn, self.half, self.rsub = me, sgn, half, rsub
        self.to, self.frm = to, frm
        self.send, self.recv, self.ld, self.acc, self.stg = bufs
        (self.ssem, self.rsem, self.ldsem, self.stsem,
         self.agss, self.agrs) = sems
        self.rscred, self.agcred = creds
        self.rdma = None

    def chunk(self, k):
        return lax.rem(self.me + self.sgn * k + 2 * N_DEV, N_DEV)


def _ar_body(part_ref, scale_ref, out_ref, *scr):
    me = lax.axis_index("i")
    right = lax.rem(me + 1, N_DEV)
    left = lax.rem(me + N_DEV - 1, N_DEV)
    ro = pl.program_id(0) * HROW

    barrier = pltpu.get_barrier_semaphore()
    for nbr in (left, right):
        pl.semaphore_signal(barrier, inc=1, device_id=(nbr,),
                            device_id_type=pl.DeviceIdType.MESH)
    pl.semaphore_wait(barrier, 2)

    cfg = ((-1, 0, 0), (+1, 1, 0), (-1, 0, 1), (+1, 1, 1))
    lanes = tuple(
        _Lane(me, sgn, half, rsub,
              right if sgn < 0 else left,
              left if sgn < 0 else right,
              scr[5 * i: 5 * i + 5],
              scr[20 + 6 * i: 20 + 6 * i + 6],
              scr[44 + 2 * i: 44 + 2 * i + 2])
        for i, (sgn, half, rsub) in enumerate(cfg)
    )

    def blk(c, ln):
        return part_ref.at[pl.ds(ro + ln.rsub * SROW, SROW),
                           pl.ds(c * CW + ln.half * CW2, CW2)]

    def oblk(c, ln):
        return out_ref.at[pl.ds(ro + ln.rsub * SROW, SROW),
                          pl.ds(c * CW + ln.half * CW2, CW2)]

    def rs_rdma(ln, s):
        return pltpu.make_async_remote_copy(
            src_ref=ln.send.at[s % 2],
            dst_ref=ln.recv.at[s % 2],
            send_sem=ln.ssem.at[s % 2],
            recv_sem=ln.rsem.at[s % 2],
            device_id=(ln.to,),
            device_id_type=pl.DeviceIdType.MESH,
        )

    def ag_rdma(ln, t):
        return pltpu.make_async_remote_copy(
            src_ref=ln.send.at[t % 2],
            dst_ref=ln.send.at[(t + 1) % 2],
            send_sem=ln.agss.at[t % 2],
            recv_sem=ln.agrs.at[(t + 1) % 2],
            device_id=(ln.to,),
            device_id_type=pl.DeviceIdType.MESH,
        )

    for ln in lanes:
        pltpu.make_async_copy(blk(me, ln), ln.ld, ln.ldsem).start()
    for ln in lanes:
        pltpu.make_async_copy(blk(me, ln), ln.ld, ln.ldsem).wait()
        ln.send[0] = ln.ld[...].astype(jnp.bfloat16)
        ln.rdma = rs_rdma(ln, 0)
        ln.rdma.start()
        pltpu.make_async_copy(blk(ln.chunk(1), ln), ln.ld, ln.ldsem).start()

    for s in range(N_DEV - 1):
        slot = s % 2
        for ln in lanes:
            pltpu.make_async_copy(blk(ln.chunk(s + 1), ln),
                                  ln.ld, ln.ldsem).wait()
            ln.rdma.wait()
            if s < N_DEV - 2:
                ln.send[(s + 1) % 2] = (
                    ln.recv[slot].astype(jnp.float32) + ln.ld[...]
                ).astype(jnp.bfloat16)
            else:
                ln.acc[...] = ln.recv[slot].astype(jnp.float32) + ln.ld[...]
            if s <= N_DEV - 4:
                pl.semaphore_signal(ln.rscred, inc=1, device_id=(ln.frm,),
                                    device_id_type=pl.DeviceIdType.MESH)
            if s == N_DEV - 3:
                pl.semaphore_signal(ln.agcred, inc=1, device_id=(ln.frm,),
                                    device_id_type=pl.DeviceIdType.MESH)
            if s < N_DEV - 2:
                if s + 1 >= 2:
                    pl.semaphore_wait(ln.rscred, 1)
                ln.rdma = rs_rdma(ln, s + 1)
                ln.rdma.start()
                pltpu.make_async_copy(blk(ln.chunk(s + 2), ln),
                                      ln.ld, ln.ldsem).start()

    for ln in lanes:
        own = ln.chunk(N_DEV - 1)
        y = ln.acc[...] * scale_ref[0, 0]
        sg = 1.0 / (1.0 + jnp.exp(-jnp.clip(y, -60.0, 60.0)))
        ln.stg[1] = y * sg
        ln.send[0] = ln.stg[1].astype(jnp.bfloat16)
        pltpu.make_async_copy(ln.stg.at[1], oblk(own, ln),
                              ln.stsem.at[1]).start()
        pl.semaphore_wait(ln.agcred, 1)
        ln.rdma = ag_rdma(ln, 0)
        ln.rdma.start()

    for t in range(N_DEV - 1):
        for ln in lanes:
            ln.rdma.wait()
            if t <= N_DEV - 3:
                pl.semaphore_signal(ln.agcred, inc=1, device_id=(ln.frm,),
                                    device_id_type=pl.DeviceIdType.MESH)
            if t < N_DEV - 2:
                pl.semaphore_wait(ln.agcred, 1)
                ln.rdma = ag_rdma(ln, t + 1)
                ln.rdma.start()
            r = ln.chunk(t)
            if t >= 1:
                pltpu.make_async_copy(ln.stg.at[t % 2], oblk(r, ln),
                                      ln.stsem.at[t % 2]).wait()
            ln.stg[t % 2] = ln.send[(t + 1) % 2].astype(jnp.float32)
            pltpu.make_async_copy(ln.stg.at[t % 2], oblk(r, ln),
                                  ln.stsem.at[t % 2]).start()
    for ln in lanes:
        for sl in (1, 0):
            pltpu.make_async_copy(ln.stg.at[sl], oblk(me, ln),
                                  ln.stsem.at[sl]).wait()


def kernel(x, w_mat, scale_x, scale_w):
    partial = jnp.dot(x.astype(jnp.bfloat16), w_mat.astype(jnp.bfloat16),
                      preferred_element_type=jnp.float32)
    scale = jnp.reshape(scale_x.astype(jnp.float32)
                        * scale_w.astype(jnp.float32), (1, 1))

    lane_bufs = [
        pltpu.VMEM((2, SROW, CW2), jnp.bfloat16),
        pltpu.VMEM((2, SROW, CW2), jnp.bfloat16),
        pltpu.VMEM((SROW, CW2), jnp.float32),
        pltpu.VMEM((SROW, CW2), jnp.float32),
        pltpu.VMEM((2, SROW, CW2), jnp.float32),
    ]
    lane_sems = [
        pltpu.SemaphoreType.DMA((2,)),
        pltpu.SemaphoreType.DMA((2,)),
        pltpu.SemaphoreType.DMA,
        pltpu.SemaphoreType.DMA((2,)),
        pltpu.SemaphoreType.DMA((2,)),
        pltpu.SemaphoreType.DMA((2,)),
    ]
    scratch = (lane_bufs * N_LANE
               + lane_sems * N_LANE
               + [pltpu.SemaphoreType.REGULAR] * (2 * N_LANE))

    return pl.pallas_call(
        _ar_body,
        grid=(2,),
        out_shape=jax.ShapeDtypeStruct((M, N), jnp.float32),
        in_specs=[
            pl.BlockSpec(memory_space=pl.ANY),
            pl.BlockSpec(memory_space=pltpu.MemorySpace.SMEM),
        ],
        out_specs=pl.BlockSpec(memory_space=pl.ANY),
        scratch_shapes=scratch,
        compiler_params=pltpu.CompilerParams(collective_id=0),
    )(partial, scale)
